# baseline (speedup 1.0000x reference)
import functools

import jax
import jax.numpy as jnp
from jax import lax
from jax.experimental import pallas as pl
from jax.experimental.pallas import tpu as pltpu
from jax.experimental.pallas import tpu_sc as plsc

_NC = 2
_NS = 16
_L = 16
_NW = _NC * _NS
_CHUNK = 128


@functools.lru_cache(maxsize=None)
def _build(B, D):
    assert B % (_NW * _L) == 0 and D % _L == 0
    bpw = B // _NW
    nch = bpw // _CHUNK
    ngr = bpw // _L

    mesh = plsc.VectorSubcoreMesh(core_axis_name="c", subcore_axis_name="s")

    @functools.partial(
        pl.kernel,
        mesh=mesh,
        compiler_params=pltpu.CompilerParams(
            needs_layout_passes=False, use_tc_tiling_on_sc=False),
        out_type=jax.ShapeDtypeStruct((B,), jnp.float32),
        scratch_types=[
            pltpu.VMEM((nch, _CHUNK), jnp.int32),
            pltpu.VMEM((bpw, D), jnp.float32),
            pltpu.VMEM((D + 1, _L), jnp.float32),
            pltpu.VMEM((bpw,), jnp.float32),
            pltpu.SemaphoreType.DMA,
        ],
    )
    def gather_dot(idx_hbm, aux_hbm, table_hbm, out_hbm,
                   idx_v, rows_v, aux_v, res_v, sem):
        wid = lax.axis_index("s") * _NC + lax.axis_index("c")
        base = wid * bpw
        pltpu.sync_copy(idx_hbm.at[wid], idx_v)
        pltpu.sync_copy(aux_hbm, aux_v)
        copies = [
            pltpu.async_copy(table_hbm.at[idx_v.at[j]],
                             rows_v.at[pl.ds(j * _CHUNK, _CHUNK)], sem)
            for j in range(nch)
        ]
        for c in copies:
            c.wait()

        lanes = lax.iota(jnp.int32, _L)

        def group(g, carry):
            row_ids = g * _L + lanes
            acc = aux_v[D, :]
            for d in range(D):
                col = plsc.load_gather(
                    rows_v, [row_ids, jnp.full((_L,), d, jnp.int32)])
                acc = acc + col * aux_v[d, :]
            res_v[pl.ds(g * _L, _L)] = acc
            return carry

        lax.fori_loop(0, ngr, group, 0)
        pltpu.sync_copy(res_v, out_hbm.at[pl.ds(base, bpw)])

    return gather_dot


def kernel(x_movie, x_user, movie_table, fc_w, fc_b):
    B = x_movie.shape[0]
    D = movie_table.shape[1]
    wc = fc_w[0, :D] + fc_w[0, D:]
    aux = jnp.broadcast_to(
        jnp.concatenate([wc, fc_b])[:, None], (D + 1, _L)).astype(jnp.float32)
    idx = x_movie.astype(jnp.int32).reshape(_NW, B // (_NW * _CHUNK), _CHUNK)
    out = _build(B, D)(idx, aux, movie_table)
    return out.reshape(B, 1)
